# baseline (speedup 1.0000x reference)
import functools

import jax
import jax.numpy as jnp
from jax import lax
from jax.experimental import pallas as pl
from jax.experimental.pallas import tpu as pltpu
from jax.experimental.pallas import tpu_sc as plsc

_MOM = 0.9


def _sc_update_builder(n_idx, d):
    mesh = plsc.VectorSubcoreMesh(core_axis_name="c", subcore_axis_name="s")
    chunks_per_w = n_idx // 32 // 128

    @functools.partial(
        pl.kernel,
        out_type=(),
        mesh=mesh,
        scratch_types=[
            pltpu.VMEM((chunks_per_w, 128), jnp.int32),
            pltpu.VMEM((chunks_per_w, 128, d), jnp.float32),
            pltpu.VMEM((d,), jnp.float32),
            pltpu.SemaphoreType.DMA,
            pltpu.SemaphoreType.DMA,
        ],
    )
    def sc_update(dst_ref, que_hbm, keys_hbm, idx_hbm, idx_v, rows_v, keys_v,
                  gsem, ssem):
        c = lax.axis_index("c")
        s = lax.axis_index("s")
        w = s * 2 + c

        pltpu.sync_copy(keys_hbm, keys_v)
        for i in range(d // 16):
            ksl = pl.ds(i * 16, 16)
            keys_v[ksl] = keys_v[ksl] * jnp.float32(1.0 - _MOM)
        pltpu.sync_copy(idx_hbm.at[w], idx_v)

        gathers = [
            pltpu.async_copy(que_hbm.at[idx_v.at[j]], rows_v.at[j], gsem)
            for j in range(chunks_per_w)
        ]
        kvecs = [keys_v[pl.ds(i * 16, 16)] for i in range(d // 16)]
        scatters = []
        for j in range(chunks_per_w):
            gathers[j].wait()

            def _blend_row(r, carry):
                for i in range(d // 16):
                    sl = pl.ds(i * 16, 16)
                    rows_v[j, r, sl] = (rows_v[j, r, sl] * jnp.float32(_MOM)
                                        + kvecs[i])
                return carry

            lax.fori_loop(0, 128, _blend_row, 0)
            scatters.append(pltpu.async_copy(
                rows_v.at[j], dst_ref.at[idx_v.at[j]], ssem))
        for sc in scatters:
            sc.wait()

    return sc_update


def _dma_copy_builder(n, d, n_chunks):
    chunk = n // n_chunks

    def _dma_copy_body(que_hbm, out_hbm, sem):
        copies = [
            pltpu.make_async_copy(
                que_hbm.at[pl.ds(k * chunk, chunk)],
                out_hbm.at[pl.ds(k * chunk, chunk)], sem)
            for k in range(n_chunks)
        ]
        for cp in copies:
            cp.start()
        for cp in copies:
            cp.wait()

    return pl.pallas_call(
        _dma_copy_body,
        in_specs=[pl.BlockSpec(memory_space=pl.ANY)],
        out_specs=pl.BlockSpec(memory_space=pl.ANY),
        out_shape=jax.ShapeDtypeStruct((n, d), jnp.float32),
        scratch_shapes=[pltpu.SemaphoreType.DMA],
    )


def kernel(que, keys, index):
    n, d = que.shape
    b = index.shape[0]

    out = _dma_copy_builder(n, d, 8)(que)

    idx3 = index.astype(jnp.int32).reshape(32, b // 32 // 128, 128)
    out_ref = jax.new_ref(out)
    _sc_update_builder(b, d)(out_ref, que, keys, idx3)
    return out_ref[...]


# speedup vs baseline: 27.8886x; 27.8886x over previous
import functools

import jax
import jax.numpy as jnp
from jax import lax
from jax.experimental import pallas as pl
from jax.experimental.pallas import tpu as pltpu
from jax.experimental.pallas import tpu_sc as plsc

_MOM = 0.9


def _sc_update_builder(n_idx, d):
    mesh = plsc.VectorSubcoreMesh(core_axis_name="c", subcore_axis_name="s")
    chunks_per_w = n_idx // 32 // 128

    @functools.partial(
        pl.kernel,
        out_type=(),
        mesh=mesh,
        scratch_types=[
            pltpu.VMEM((chunks_per_w, 128), jnp.int32),
            pltpu.VMEM((chunks_per_w, 128, d), jnp.float32),
            pltpu.VMEM((d,), jnp.float32),
            pltpu.SemaphoreType.DMA,
            pltpu.SemaphoreType.DMA,
        ],
    )
    def sc_update(dst_ref, que_hbm, keys_hbm, idx_hbm, idx_v, rows_v, keys_v,
                  gsem, ssem):
        c = lax.axis_index("c")
        s = lax.axis_index("s")
        w = s * 2 + c

        pltpu.sync_copy(keys_hbm, keys_v)
        for i in range(d // 16):
            ksl = pl.ds(i * 16, 16)
            keys_v[ksl] = keys_v[ksl] * jnp.float32(1.0 - _MOM)
        pltpu.sync_copy(idx_hbm.at[w], idx_v)

        gathers = [
            pltpu.async_copy(que_hbm.at[idx_v.at[j]], rows_v.at[j], gsem)
            for j in range(chunks_per_w)
        ]
        kvecs = [keys_v[pl.ds(i * 16, 16)] for i in range(d // 16)]
        scatters = []
        for j in range(chunks_per_w):
            gathers[j].wait()

            def _blend_row(r, carry):
                for i in range(d // 16):
                    sl = pl.ds(i * 16, 16)
                    rows_v[j, r, sl] = (rows_v[j, r, sl] * jnp.float32(_MOM)
                                        + kvecs[i])
                return carry

            lax.fori_loop(0, 128, _blend_row, 0)
            scatters.append(pltpu.async_copy(
                rows_v.at[j], dst_ref.at[idx_v.at[j]], ssem))
        for sc in scatters:
            sc.wait()

    return sc_update


def _copy_body(que_ref, out_ref):
    out_ref[...] = que_ref[...]


def kernel(que, keys, index):
    n, d = que.shape
    b = index.shape[0]

    block_rows = 20000
    out = pl.pallas_call(
        _copy_body,
        grid=(n // block_rows,),
        in_specs=[pl.BlockSpec((block_rows, d), lambda i: (i, 0))],
        out_specs=pl.BlockSpec((block_rows, d), lambda i: (i, 0)),
        out_shape=jax.ShapeDtypeStruct((n, d), jnp.float32),
    )(que)

    idx3 = index.astype(jnp.int32).reshape(32, b // 32 // 128, 128)
    out_ref = jax.new_ref(out)
    _sc_update_builder(b, d)(out_ref, que, keys, idx3)
    return out_ref[...]


# speedup vs baseline: 27.9476x; 1.0021x over previous
import functools

import jax
import jax.numpy as jnp
from jax import lax
from jax.experimental import pallas as pl
from jax.experimental.pallas import tpu as pltpu
from jax.experimental.pallas import tpu_sc as plsc

_MOM = 0.9


def _sc_update_builder(n_idx, d):
    mesh = plsc.VectorSubcoreMesh(core_axis_name="c", subcore_axis_name="s")
    chunks_per_w = n_idx // 32 // 128

    @functools.partial(
        pl.kernel,
        out_type=(),
        mesh=mesh,
        scratch_types=[
            pltpu.VMEM((chunks_per_w, 128), jnp.int32),
            pltpu.VMEM((chunks_per_w, 128, d), jnp.float32),
            pltpu.VMEM((d,), jnp.float32),
            pltpu.SemaphoreType.DMA,
            pltpu.SemaphoreType.DMA,
        ],
    )
    def sc_update(dst_ref, que_hbm, keys_hbm, idx_hbm, idx_v, rows_v, keys_v,
                  gsem, ssem):
        c = lax.axis_index("c")
        s = lax.axis_index("s")
        w = s * 2 + c

        pltpu.sync_copy(keys_hbm, keys_v)
        for i in range(d // 16):
            ksl = pl.ds(i * 16, 16)
            keys_v[ksl] = keys_v[ksl] * jnp.float32(1.0 - _MOM)
        pltpu.sync_copy(idx_hbm.at[w], idx_v)

        gathers = [
            pltpu.async_copy(que_hbm.at[idx_v.at[j]], rows_v.at[j], gsem)
            for j in range(chunks_per_w)
        ]
        kvecs = [keys_v[pl.ds(i * 16, 16)] for i in range(d // 16)]
        scatters = []
        for j in range(chunks_per_w):
            gathers[j].wait()

            def _blend_row(r, carry):
                for i in range(d // 16):
                    sl = pl.ds(i * 16, 16)
                    rows_v[j, r, sl] = (rows_v[j, r, sl] * jnp.float32(_MOM)
                                        + kvecs[i])
                return carry

            lax.fori_loop(0, 128, _blend_row, 0)
            scatters.append(pltpu.async_copy(
                rows_v.at[j], dst_ref.at[idx_v.at[j]], ssem))
        for sc in scatters:
            sc.wait()

    return sc_update


def _copy_body(que_ref, out_ref):
    out_ref[...] = que_ref[...]


def kernel(que, keys, index):
    n, d = que.shape
    b = index.shape[0]

    block_rows = 20000
    out = pl.pallas_call(
        _copy_body,
        grid=(n // block_rows,),
        in_specs=[pl.BlockSpec((block_rows, d), lambda i: (i, 0))],
        out_specs=pl.BlockSpec((block_rows, d), lambda i: (i, 0)),
        out_shape=jax.ShapeDtypeStruct((n, d), jnp.float32),
    )(que)

    idx3 = index.astype(jnp.int32).reshape(32, b // 32 // 128, 128)
    out_ref = jax.new_ref(out)
    _sc_update_builder(b, d)(out_ref, que, keys, idx3)
    return jax.freeze(out_ref)
